# baseline (speedup 1.0000x reference)
import functools

import jax
import jax.numpy as jnp
from jax import lax
from jax.experimental import pallas as pl
from jax.experimental.pallas import tpu as pltpu
from jax.experimental.pallas import tpu_sc as plsc

N_LEVELS = 10
F = 16
N = 100000
OUT_D = F * N_LEVELS

C = 128
NFULL = N // C
NUNITS = NFULL + 1
TAIL_BASE = N - C
NW = 32
LANES = 16


def _reblock_body(x_ref, o_ref):
    o_ref[...] = x_ref[...].reshape(o_ref.shape)


def _reblock(arr):
    return pl.pallas_call(
        _reblock_body,
        grid=(N_LEVELS, NFULL),
        in_specs=[pl.BlockSpec((1, 4, C), lambda l, c: (l, 0, c))],
        out_specs=pl.BlockSpec((1, 1, 4, C), lambda l, c: (c, l, 0, 0)),
        out_shape=jax.ShapeDtypeStruct((NFULL, N_LEVELS, 4, C), arr.dtype),
    )(arr)


def _heal_body(pix_hbm, w_hbm, tpix_hbm, tw_hbm, table_hbm, out_hbm,
               pix_v, w_v, idx_v, rows_v, out_v, sem_g0, sem_g1):
    wid = lax.axis_index("s") * 2 + lax.axis_index("c")
    niter = (NUNITS - wid + NW - 1) // NW

    iota = lax.iota(jnp.int32, LANES)
    col_base = iota * N_LEVELS

    def compute_level(l, rbuf):
        col = col_base + l

        def group(g, _):
            b16 = g * LANES
            wv = [w_v[l, j, pl.ds(b16, LANES)] for j in range(4)]
            for p in range(LANES):
                n = b16 + p
                acc = wv[0][p] * rbuf[n]
                acc = acc + wv[1][p] * rbuf[C + n]
                acc = acc + wv[2][p] * rbuf[2 * C + n]
                acc = acc + wv[3][p] * rbuf[3 * C + n]
                row_ids = jnp.full((LANES,), n, dtype=jnp.int32)
                plsc.store_scatter(out_v, [row_ids, col], acc)
            return 0
        lax.fori_loop(0, C // LANES, group, 0)

    def unit_body(it, _):
        u = wid + it * NW
        is_tail = u >= NFULL
        base = jnp.where(is_tail, TAIL_BASE, u * C)

        @pl.when(jnp.logical_not(is_tail))
        def _():
            pltpu.sync_copy(pix_hbm.at[u], pix_v)
            pltpu.sync_copy(w_hbm.at[u], w_v)

        @pl.when(is_tail)
        def _():
            pltpu.sync_copy(tpix_hbm, pix_v)
            pltpu.sync_copy(tw_hbm, w_v)

        def idx_level(l, _):
            start = ((jnp.int32(1) << (2 * l)) - 1) * 4

            def ib(i, _):
                for j in range(4):
                    v = pix_v[l, j, pl.ds(i * LANES, LANES)]
                    idx_v[l, pl.ds(j * C + i * LANES, LANES)] = v + start
                return 0
            lax.fori_loop(0, C // LANES, ib, 0)
            return 0
        lax.fori_loop(0, N_LEVELS, idx_level, 0)

        pltpu.async_copy(table_hbm.at[idx_v.at[0]], rows_v.at[0], sem_g0)

        def pair(i, _):
            l0 = 2 * i
            pltpu.async_copy(table_hbm.at[idx_v.at[l0 + 1]], rows_v.at[1],
                             sem_g1)
            pltpu.make_async_copy(table_hbm.at[idx_v.at[l0]], rows_v.at[0],
                                  sem_g0).wait()
            compute_level(l0, rows_v.at[0])

            @pl.when(i < (N_LEVELS // 2 - 1))
            def _():
                pltpu.async_copy(table_hbm.at[idx_v.at[l0 + 2]],
                                 rows_v.at[0], sem_g0)

            pltpu.make_async_copy(table_hbm.at[idx_v.at[l0 + 1]],
                                  rows_v.at[1], sem_g1).wait()
            compute_level(l0 + 1, rows_v.at[1])
            return 0
        lax.fori_loop(0, N_LEVELS // 2, pair, 0)

        pltpu.sync_copy(out_v, out_hbm.at[pl.ds(base, C)])
        return 0

    lax.fori_loop(0, niter, unit_body, 0)


def _heal_sc(params, neigh_pix, neigh_weight):
    mesh = plsc.VectorSubcoreMesh(core_axis_name="c", subcore_axis_name="s")
    kfn = pl.kernel(
        _heal_body,
        mesh=mesh,
        out_type=jax.ShapeDtypeStruct((N, OUT_D), jnp.float32),
        scratch_types=[
            pltpu.VMEM((N_LEVELS, 4, C), jnp.int32),
            pltpu.VMEM((N_LEVELS, 4, C), jnp.float32),
            pltpu.VMEM((N_LEVELS, 4 * C), jnp.int32),
            pltpu.VMEM((2, 4 * C, F), jnp.float32),
            pltpu.VMEM((C, OUT_D), jnp.float32),
            pltpu.SemaphoreType.DMA,
            pltpu.SemaphoreType.DMA,
        ],
        compiler_params=pltpu.CompilerParams(
            use_tc_tiling_on_sc=False, needs_layout_passes=False),
    )
    pix_blk = _reblock(neigh_pix)
    w_blk = _reblock(neigh_weight)
    tail_pix = neigh_pix[:, :, TAIL_BASE:]
    tail_w = neigh_weight[:, :, TAIL_BASE:]
    return kfn(pix_blk, w_blk, tail_pix, tail_w, params)


def kernel(x, params, neigh_pix, neigh_weight):
    del x
    return _heal_sc(params, neigh_pix, neigh_weight)


# speedup vs baseline: 2.8318x; 2.8318x over previous
import functools

import jax
import jax.numpy as jnp
from jax import lax
from jax.experimental import pallas as pl
from jax.experimental.pallas import tpu as pltpu
from jax.experimental.pallas import tpu_sc as plsc

N_LEVELS = 10
F = 16
N = 100000
OUT_D = F * N_LEVELS

C = 128
NFULL = N // C
NUNITS = NFULL + 1
TAIL_BASE = N - C
NW = 32
LANES = 16


NCB = 71


def _reblock_body(x_ref, o_ref):
    x = x_ref[...].reshape(4, NCB, C)
    for j in range(4):
        o_ref[:, 0, j, :] = x[j]


def _reblock(arr):
    return pl.pallas_call(
        _reblock_body,
        grid=(N_LEVELS, NFULL // NCB),
        in_specs=[pl.BlockSpec((1, 4, NCB * C), lambda l, c: (l, 0, c))],
        out_specs=pl.BlockSpec((NCB, 1, 4, C), lambda l, c: (c, l, 0, 0)),
        out_shape=jax.ShapeDtypeStruct((NFULL, N_LEVELS, 4, C), arr.dtype),
    )(arr)


def _heal_body(pix_hbm, w_hbm, tpix_hbm, tw_hbm, table_hbm, out_hbm,
               pix_v, w_v, idx_v, rows_v, out_v, sem_g0, sem_g1):
    wid = lax.axis_index("s") * 2 + lax.axis_index("c")
    niter = (NUNITS - wid + NW - 1) // NW

    iota = lax.iota(jnp.int32, LANES)
    col_base = iota * N_LEVELS

    def compute_level(l, rbuf):
        col = col_base + l

        def group(g, _):
            b16 = g * LANES
            wv = [w_v[l, j, pl.ds(b16, LANES)] for j in range(4)]
            for p in range(LANES):
                n = b16 + p
                acc = wv[0][p] * rbuf[n]
                acc = acc + wv[1][p] * rbuf[C + n]
                acc = acc + wv[2][p] * rbuf[2 * C + n]
                acc = acc + wv[3][p] * rbuf[3 * C + n]
                row_ids = jnp.full((LANES,), n, dtype=jnp.int32)
                plsc.store_scatter(out_v, [row_ids, col], acc)
            return 0
        lax.fori_loop(0, C // LANES, group, 0)

    def unit_body(it, _):
        u = wid + it * NW
        is_tail = u >= NFULL
        base = jnp.where(is_tail, TAIL_BASE, u * C)

        @pl.when(jnp.logical_not(is_tail))
        def _():
            pltpu.sync_copy(pix_hbm.at[u], pix_v)
            pltpu.sync_copy(w_hbm.at[u], w_v)

        @pl.when(is_tail)
        def _():
            pltpu.sync_copy(tpix_hbm, pix_v)
            pltpu.sync_copy(tw_hbm, w_v)

        def idx_level(l, _):
            start = ((jnp.int32(1) << (2 * l)) - 1) * 4

            def ib(i, _):
                for j in range(4):
                    v = pix_v[l, j, pl.ds(i * LANES, LANES)]
                    idx_v[l, pl.ds(j * C + i * LANES, LANES)] = v + start
                return 0
            lax.fori_loop(0, C // LANES, ib, 0)
            return 0
        lax.fori_loop(0, N_LEVELS, idx_level, 0)

        pltpu.async_copy(table_hbm.at[idx_v.at[0]], rows_v.at[0], sem_g0)

        def pair(i, _):
            l0 = 2 * i
            pltpu.async_copy(table_hbm.at[idx_v.at[l0 + 1]], rows_v.at[1],
                             sem_g1)
            pltpu.make_async_copy(table_hbm.at[idx_v.at[l0]], rows_v.at[0],
                                  sem_g0).wait()
            compute_level(l0, rows_v.at[0])

            @pl.when(i < (N_LEVELS // 2 - 1))
            def _():
                pltpu.async_copy(table_hbm.at[idx_v.at[l0 + 2]],
                                 rows_v.at[0], sem_g0)

            pltpu.make_async_copy(table_hbm.at[idx_v.at[l0 + 1]],
                                  rows_v.at[1], sem_g1).wait()
            compute_level(l0 + 1, rows_v.at[1])
            return 0
        lax.fori_loop(0, N_LEVELS // 2, pair, 0)

        pltpu.sync_copy(out_v, out_hbm.at[pl.ds(base, C)])
        return 0

    lax.fori_loop(0, niter, unit_body, 0)


def _heal_sc(params, neigh_pix, neigh_weight):
    mesh = plsc.VectorSubcoreMesh(core_axis_name="c", subcore_axis_name="s")
    kfn = pl.kernel(
        _heal_body,
        mesh=mesh,
        out_type=jax.ShapeDtypeStruct((N, OUT_D), jnp.float32),
        scratch_types=[
            pltpu.VMEM((N_LEVELS, 4, C), jnp.int32),
            pltpu.VMEM((N_LEVELS, 4, C), jnp.float32),
            pltpu.VMEM((N_LEVELS, 4 * C), jnp.int32),
            pltpu.VMEM((2, 4 * C, F), jnp.float32),
            pltpu.VMEM((C, OUT_D), jnp.float32),
            pltpu.SemaphoreType.DMA,
            pltpu.SemaphoreType.DMA,
        ],
        compiler_params=pltpu.CompilerParams(
            use_tc_tiling_on_sc=False, needs_layout_passes=False),
    )
    pix_blk = _reblock(neigh_pix)
    w_blk = _reblock(neigh_weight)
    tail_pix = neigh_pix[:, :, TAIL_BASE:]
    tail_w = neigh_weight[:, :, TAIL_BASE:]
    return kfn(pix_blk, w_blk, tail_pix, tail_w, params)


def kernel(x, params, neigh_pix, neigh_weight):
    del x
    return _heal_sc(params, neigh_pix, neigh_weight)


# speedup vs baseline: 2.8341x; 1.0008x over previous
import functools

import jax
import jax.numpy as jnp
from jax import lax
from jax.experimental import pallas as pl
from jax.experimental.pallas import tpu as pltpu
from jax.experimental.pallas import tpu_sc as plsc

N_LEVELS = 10
F = 16
N = 100000
OUT_D = F * N_LEVELS

C = 128
NFULL = N // C
NUNITS = NFULL + 1
TAIL_BASE = N - C
NW = 32
LANES = 16


CW = 4 * N_LEVELS * C
NP_FULL = NFULL * C


def _reblock_body(x_ref, o_ref):
    x = x_ref[...].reshape(4, NFULL, C)
    for j in range(4):
        o_ref[:, j * C:(j + 1) * C] = x[j]


def _reblock(arr):
    return pl.pallas_call(
        _reblock_body,
        grid=(N_LEVELS,),
        in_specs=[pl.BlockSpec((1, 4, NP_FULL), lambda l: (l, 0, 0))],
        out_specs=pl.BlockSpec((NFULL, 4 * C), lambda l: (0, l)),
        out_shape=jax.ShapeDtypeStruct((NFULL, CW), arr.dtype),
    )(arr[:, :, :NP_FULL])


def _heal_body(pix_hbm, w_hbm, tpix_hbm, tw_hbm, table_hbm, out_hbm,
               pix_v, w_v, idx_v, rows_v, out_v, sem_g0, sem_g1):
    wid = lax.axis_index("s") * 2 + lax.axis_index("c")
    niter = (NUNITS - wid + NW - 1) // NW

    iota = lax.iota(jnp.int32, LANES)
    col_base = iota * N_LEVELS

    def compute_level(l, rbuf):
        col = col_base + l
        wbase = l * 4 * C

        def group(g, _):
            b16 = g * LANES
            wv = [w_v[pl.ds(wbase + j * C + b16, LANES)] for j in range(4)]
            for p in range(LANES):
                n = b16 + p
                acc = wv[0][p] * rbuf[n]
                acc = acc + wv[1][p] * rbuf[C + n]
                acc = acc + wv[2][p] * rbuf[2 * C + n]
                acc = acc + wv[3][p] * rbuf[3 * C + n]
                row_ids = jnp.full((LANES,), n, dtype=jnp.int32)
                plsc.store_scatter(out_v, [row_ids, col], acc)
            return 0
        lax.fori_loop(0, C // LANES, group, 0)

    def unit_body(it, _):
        u = wid + it * NW
        is_tail = u >= NFULL
        base = jnp.where(is_tail, TAIL_BASE, u * C)

        @pl.when(jnp.logical_not(is_tail))
        def _():
            pltpu.sync_copy(pix_hbm.at[u], pix_v)
            pltpu.sync_copy(w_hbm.at[u], w_v)

        @pl.when(is_tail)
        def _():
            pltpu.sync_copy(tpix_hbm, pix_v)
            pltpu.sync_copy(tw_hbm, w_v)

        def idx_level(l, _):
            start = ((jnp.int32(1) << (2 * l)) - 1) * 4
            pbase = l * 4 * C

            def ib(i, _):
                for j in range(4):
                    v = pix_v[pl.ds(pbase + j * C + i * LANES, LANES)]
                    idx_v[l, pl.ds(j * C + i * LANES, LANES)] = v + start
                return 0
            lax.fori_loop(0, C // LANES, ib, 0)
            return 0
        lax.fori_loop(0, N_LEVELS, idx_level, 0)

        pltpu.async_copy(table_hbm.at[idx_v.at[0]], rows_v.at[0], sem_g0)

        def pair(i, _):
            l0 = 2 * i
            pltpu.async_copy(table_hbm.at[idx_v.at[l0 + 1]], rows_v.at[1],
                             sem_g1)
            pltpu.make_async_copy(table_hbm.at[idx_v.at[l0]], rows_v.at[0],
                                  sem_g0).wait()
            compute_level(l0, rows_v.at[0])

            @pl.when(i < (N_LEVELS // 2 - 1))
            def _():
                pltpu.async_copy(table_hbm.at[idx_v.at[l0 + 2]],
                                 rows_v.at[0], sem_g0)

            pltpu.make_async_copy(table_hbm.at[idx_v.at[l0 + 1]],
                                  rows_v.at[1], sem_g1).wait()
            compute_level(l0 + 1, rows_v.at[1])
            return 0
        lax.fori_loop(0, N_LEVELS // 2, pair, 0)

        pltpu.sync_copy(out_v, out_hbm.at[pl.ds(base, C)])
        return 0

    lax.fori_loop(0, niter, unit_body, 0)


def _heal_sc(params, neigh_pix, neigh_weight):
    mesh = plsc.VectorSubcoreMesh(core_axis_name="c", subcore_axis_name="s")
    kfn = pl.kernel(
        _heal_body,
        mesh=mesh,
        out_type=jax.ShapeDtypeStruct((N, OUT_D), jnp.float32),
        scratch_types=[
            pltpu.VMEM((CW,), jnp.int32),
            pltpu.VMEM((CW,), jnp.float32),
            pltpu.VMEM((N_LEVELS, 4 * C), jnp.int32),
            pltpu.VMEM((2, 4 * C, F), jnp.float32),
            pltpu.VMEM((C, OUT_D), jnp.float32),
            pltpu.SemaphoreType.DMA,
            pltpu.SemaphoreType.DMA,
        ],
        compiler_params=pltpu.CompilerParams(
            use_tc_tiling_on_sc=False, needs_layout_passes=False),
    )
    pix_blk = _reblock(neigh_pix)
    w_blk = _reblock(neigh_weight)
    tail_pix = neigh_pix[:, :, TAIL_BASE:].reshape(-1)
    tail_w = neigh_weight[:, :, TAIL_BASE:].reshape(-1)
    return kfn(pix_blk, w_blk, tail_pix, tail_w, params)


def kernel(x, params, neigh_pix, neigh_weight):
    del x
    return _heal_sc(params, neigh_pix, neigh_weight)
